# baseline (speedup 1.0000x reference)
import functools

import jax
import jax.numpy as jnp
from jax.experimental import pallas as pl
from jax.experimental.pallas import tpu as pltpu

_NBUF = 3
_BM = 400


def _gcn_body(a_hbm, x_ref, w_ref, o_ref, a_buf, sems, *, nsteps):
    def copy_in(i, slot):
        return pltpu.make_async_copy(
            a_hbm.at[pl.ds(i * _BM, _BM), :], a_buf.at[slot], sems.at[slot]
        )

    for j in range(_NBUF):
        copy_in(j, j).start()

    w_row = w_ref[...]
    for i in range(nsteps):
        slot = i % _NBUF
        copy_in(i, slot).wait()
        acc = jnp.dot(a_buf[slot], x_ref[...], preferred_element_type=jnp.float32)
        o_ref[i * _BM:(i + 1) * _BM, :] = acc * w_row
        if i + _NBUF < nsteps:
            copy_in(i + _NBUF, slot).start()


def kernel(input, A, W):
    n, d = input.shape
    w2d = W.reshape(1, d)
    return pl.pallas_call(
        functools.partial(_gcn_body, nsteps=n // _BM),
        in_specs=[
            pl.BlockSpec(memory_space=pltpu.MemorySpace.HBM),
            pl.BlockSpec(memory_space=pltpu.MemorySpace.VMEM),
            pl.BlockSpec(memory_space=pltpu.MemorySpace.VMEM),
        ],
        out_specs=pl.BlockSpec(memory_space=pltpu.MemorySpace.VMEM),
        out_shape=jax.ShapeDtypeStruct((n, d), jnp.float32),
        scratch_shapes=[
            pltpu.VMEM((_NBUF, _BM, n), jnp.float32),
            pltpu.SemaphoreType.DMA((_NBUF,)),
        ],
    )(A, input, w2d)


# speedup vs baseline: 1.0257x; 1.0257x over previous
import jax
import jax.numpy as jnp
from jax.experimental import pallas as pl
from jax.experimental.pallas import tpu as pltpu


def _gcn_body(a_ref, x_ref, w_ref, o_ref):
    acc = jnp.dot(a_ref[...], x_ref[...], preferred_element_type=jnp.float32)
    o_ref[...] = acc * w_ref[...]


def kernel(input, A, W):
    n, d = input.shape
    bm = 400
    w2d = W.reshape(1, d)
    return pl.pallas_call(
        _gcn_body,
        grid=(n // bm,),
        in_specs=[
            pl.BlockSpec((bm, n), lambda m: (m, 0)),
            pl.BlockSpec((n, d), lambda m: (0, 0)),
            pl.BlockSpec((1, d), lambda m: (0, 0)),
        ],
        out_specs=pl.BlockSpec((bm, d), lambda m: (m, 0)),
        out_shape=jax.ShapeDtypeStruct((n, d), jnp.float32),
        compiler_params=pltpu.CompilerParams(
            dimension_semantics=("parallel",),
        ),
    )(A, input, w2d)
